# baseline (speedup 1.0000x reference)
import functools

import jax
import jax.numpy as jnp
from jax.experimental import pallas as pl

_N = 10000
_E = 320000
_H = 128

_EBLK = 512


def _edgewise_body(dhd_ref, ehs_ref, bhs_ref, ce_ref, ein_ref, sn_ref,
                   enew_ref, sig_ref, msg_ref):
    e_hat = dhd_ref[...] + ehs_ref[...] + ce_ref[...]
    sig = jax.nn.sigmoid(e_hat)
    sig_ref[...] = sig
    msg_ref[...] = sig * bhs_ref[...]
    enew_ref[...] = ein_ref[...] + jax.nn.relu(e_hat * sn_ref[...])


def _edgewise(dh_dst, eh_src, bh_src, ce, e_in, snorm_e):
    n_blk = _E // _EBLK
    spec = pl.BlockSpec((_EBLK, _H), lambda i: (i, 0))
    spec1 = pl.BlockSpec((_EBLK, 1), lambda i: (i, 0))
    out_shapes = [jax.ShapeDtypeStruct((_E, _H), jnp.float32)] * 3
    return pl.pallas_call(
        _edgewise_body,
        grid=(n_blk,),
        in_specs=[spec, spec, spec, spec, spec, spec1],
        out_specs=[spec, spec, spec],
        out_shape=out_shapes,
    )(dh_dst, eh_src, bh_src, ce, e_in, snorm_e)


def kernel(h, e, snorm_n, snorm_e, W_emb_h, W_emb_e, W_layers, W_ro, W_pred,
           b_pred, edge_index):
    src = edge_index[0]
    dst = edge_index[1]
    h = h @ W_emb_h
    e = e @ W_emb_e
    for l in range(4):
        h_in = h
        e_in = e
        A, B, C, Dw, Ew = (W_layers[l, i] for i in range(5))
        Ah = h @ A
        Bh = h @ B
        Dh = h @ Dw
        Eh = h @ Ew
        Ce = e @ C
        dh_dst = jnp.take(Dh, dst, axis=0)
        eh_src = jnp.take(Eh, src, axis=0)
        bh_src = jnp.take(Bh, src, axis=0)
        e_new, sig, msg = _edgewise(dh_dst, eh_src, bh_src, Ce, e_in, snorm_e)
        num = jax.ops.segment_sum(msg, dst, num_segments=_N)
        den = jax.ops.segment_sum(sig, dst, num_segments=_N)
        h_new = jax.nn.relu((Ah + num / (den + 1e-6)) * snorm_n)
        e = e_new
        h = h_in + h_new
    hro = h @ W_ro
    hg = jnp.sum(hro, axis=0, keepdims=True)
    return hg @ W_pred + b_pred


# speedup vs baseline: 2.8447x; 2.8447x over previous
import functools

import jax
import jax.numpy as jnp
from jax import lax
from jax.experimental import pallas as pl
from jax.experimental.pallas import tpu as pltpu
from jax.experimental.pallas import tpu_sc as plsc

_N = 10000
_E = 320000
_H = 128

_EBLK = 512
_CH = 128
_NCHUNK = _E // _CH
_NTILES = 32
_NPAD = 10240
_ZR = 128

_mesh = plsc.VectorSubcoreMesh(core_axis_name="c", subcore_axis_name="s")


def _zero_vmem(buf):
    z = jnp.zeros((16,), jnp.float32)

    @pl.loop(0, buf.shape[0])
    def _(r):
        @pl.loop(0, buf.shape[1], step=16)
        def _(c):
            buf[r, pl.ds(c, 16)] = z


def _gather_body(tsrc_hbm, tdst_hbm, src_hbm, dst_hbm, gsrc_hbm, gdst_hbm,
                 idx_s, idx_d, buf_s, buf_d, sem_s, sem_d):
    cid = lax.axis_index("c")
    sid = lax.axis_index("s")
    wid = sid * 2 + cid
    nloc = (_NCHUNK - wid + _NTILES - 1) // _NTILES

    def step(i, carry):
        base = (wid + i * _NTILES) * _CH
        pltpu.sync_copy(src_hbm.at[pl.ds(base, _CH)], idx_s)
        pltpu.sync_copy(dst_hbm.at[pl.ds(base, _CH)], idx_d)
        cp1 = pltpu.async_copy(tsrc_hbm.at[idx_s], buf_s, sem_s)
        cp2 = pltpu.async_copy(tdst_hbm.at[idx_d], buf_d, sem_d)
        cp1.wait()
        cp2.wait()
        pltpu.sync_copy(buf_s, gsrc_hbm.at[pl.ds(base, _CH)])
        pltpu.sync_copy(buf_d, gdst_hbm.at[pl.ds(base, _CH)])
        return carry

    lax.fori_loop(0, nloc, step, 0)


@jax.jit
def _sc_gather(table_src, table_dst, src, dst):
    return pl.kernel(
        _gather_body,
        out_type=[jax.ShapeDtypeStruct((_E, 256), jnp.float32),
                  jax.ShapeDtypeStruct((_E, _H), jnp.float32)],
        mesh=_mesh,
        scratch_types=[
            pltpu.VMEM((_CH,), jnp.int32),
            pltpu.VMEM((_CH,), jnp.int32),
            pltpu.VMEM((_CH, 256), jnp.float32),
            pltpu.VMEM((_CH, _H), jnp.float32),
            pltpu.SemaphoreType.DMA,
            pltpu.SemaphoreType.DMA,
        ],
    )(table_src, table_dst, src, dst)


def _scatter_loop(d_hbm, dst_hbm, acc, dbuf, idx, sid):
    nloc = (_NCHUNK - sid + 15) // 16

    def step(i, carry):
        base = (sid + i * 16) * _CH
        pltpu.sync_copy(d_hbm.at[pl.ds(base, _CH)], dbuf)
        pltpu.sync_copy(dst_hbm.at[pl.ds(base, _CH)], idx)
        pltpu.sync_copy(dbuf, acc.at[idx], add=True)
        return carry

    lax.fori_loop(0, nloc, step, 0)


def _scatter_body(msg_hbm, sig_hbm, dst_hbm, num_hbm, den_hbm,
                  acc, zbuf, dbuf, idx):
    cid = lax.axis_index("c")
    sid = lax.axis_index("s")
    _zero_vmem(zbuf)
    row0 = sid * (_NPAD // 16)

    @pl.loop(0, _NPAD // 16, step=_ZR)
    def _(k):
        pltpu.sync_copy(zbuf, acc.at[pl.ds(row0 + k, _ZR)])

    plsc.subcore_barrier()

    @pl.when(cid == 0)
    def _():
        _scatter_loop(msg_hbm, dst_hbm, acc, dbuf, idx, sid)

    @pl.when(cid == 1)
    def _():
        _scatter_loop(sig_hbm, dst_hbm, acc, dbuf, idx, sid)

    plsc.subcore_barrier()

    @pl.when(cid == 0)
    def _():
        @pl.loop(0, _NPAD // 16, step=_ZR)
        def _(k):
            pltpu.sync_copy(acc.at[pl.ds(row0 + k, _ZR)],
                            num_hbm.at[pl.ds(row0 + k, _ZR)])

    @pl.when(cid == 1)
    def _():
        @pl.loop(0, _NPAD // 16, step=_ZR)
        def _(k):
            pltpu.sync_copy(acc.at[pl.ds(row0 + k, _ZR)],
                            den_hbm.at[pl.ds(row0 + k, _ZR)])


@jax.jit
def _sc_scatter(msg, sig, dst):
    return pl.kernel(
        _scatter_body,
        out_type=[jax.ShapeDtypeStruct((_NPAD, _H), jnp.float32),
                  jax.ShapeDtypeStruct((_NPAD, _H), jnp.float32)],
        mesh=_mesh,
        scratch_types=[
            pltpu.VMEM_SHARED((_NPAD, _H), jnp.float32),
            pltpu.VMEM((_ZR, _H), jnp.float32),
            pltpu.VMEM((_CH, _H), jnp.float32),
            pltpu.VMEM((_CH,), jnp.int32),
        ],
    )(msg, sig, dst)


def _edgewise_body(gs_ref, gd_ref, ce_ref, ein_ref, sn_ref,
                   enew_ref, sig_ref, msg_ref):
    eh_src = gs_ref[:, :_H]
    bh_src = gs_ref[:, _H:]
    e_hat = gd_ref[...] + eh_src + ce_ref[...]
    sig = jax.nn.sigmoid(e_hat)
    sig_ref[...] = sig
    msg_ref[...] = sig * bh_src
    enew_ref[...] = ein_ref[...] + jax.nn.relu(e_hat * sn_ref[...])


def _edgewise(g_src, g_dst, ce, e_in, snorm_e):
    n_blk = _E // _EBLK
    spec = pl.BlockSpec((_EBLK, _H), lambda i: (i, 0))
    spec2 = pl.BlockSpec((_EBLK, 256), lambda i: (i, 0))
    spec1 = pl.BlockSpec((_EBLK, 1), lambda i: (i, 0))
    out_shapes = [jax.ShapeDtypeStruct((_E, _H), jnp.float32)] * 3
    return pl.pallas_call(
        _edgewise_body,
        grid=(n_blk,),
        in_specs=[spec2, spec, spec, spec, spec1],
        out_specs=[spec, spec, spec],
        out_shape=out_shapes,
    )(g_src, g_dst, ce, e_in, snorm_e)


def kernel(h, e, snorm_n, snorm_e, W_emb_h, W_emb_e, W_layers, W_ro, W_pred,
           b_pred, edge_index):
    src = edge_index[0]
    dst = edge_index[1]
    h = h @ W_emb_h
    e = e @ W_emb_e
    for l in range(4):
        h_in = h
        e_in = e
        A, B, C, Dw, Ew = (W_layers[l, i] for i in range(5))
        Ah = h @ A
        Bh = h @ B
        Dh = h @ Dw
        Eh = h @ Ew
        Ce = e @ C
        table_src = jnp.concatenate([Eh, Bh], axis=1)
        g_src, g_dst = _sc_gather(table_src, Dh, src, dst)
        e_new, sig, msg = _edgewise(g_src, g_dst, Ce, e_in, snorm_e)
        num, den = _sc_scatter(msg, sig, dst)
        num = num[:_N]
        den = den[:_N]
        h_new = jax.nn.relu((Ah + num / (den + 1e-6)) * snorm_n)
        e = e_new
        h = h_in + h_new
    hro = h @ W_ro
    hg = jnp.sum(hro, axis=0, keepdims=True)
    return hg @ W_pred + b_pred


# speedup vs baseline: 3.3449x; 1.1758x over previous
import functools

import jax
import jax.numpy as jnp
from jax import lax
from jax.experimental import pallas as pl
from jax.experimental.pallas import tpu as pltpu
from jax.experimental.pallas import tpu_sc as plsc

_N = 10000
_E = 320000
_H = 128

_EBLK = 512
_CH = 128
_NCHUNK = _E // _CH
_NTILES = 32
_NPAD = 10240
_ZR = 128

_GPT = 80
_SPT = 160

_mesh = plsc.VectorSubcoreMesh(core_axis_name="c", subcore_axis_name="s")


def _zero_vmem(buf):
    z = jnp.zeros((16,), jnp.float32)

    @pl.loop(0, buf.shape[0])
    def _(r):
        @pl.loop(0, buf.shape[1], step=16)
        def _(c):
            buf[r, pl.ds(c, 16)] = z


def _gather_body(tsrc_hbm, tdst_hbm, src2d_hbm, dst2d_hbm, g_hbm,
                 idx_s, idx_d, bs0, bs1, bd0, bd1,
                 sgs0, sgs1, sgd0, sgd1, sw0, sw1):
    cid = lax.axis_index("c")
    sid = lax.axis_index("s")
    wid = sid * 2 + cid
    r0 = wid * _GPT
    count = jnp.minimum(_GPT, _NCHUNK - r0)

    @pl.when(wid < _NTILES - 1)
    def _():
        pltpu.sync_copy(src2d_hbm.at[pl.ds(r0, _GPT)], idx_s)
        pltpu.sync_copy(dst2d_hbm.at[pl.ds(r0, _GPT)], idx_d)

    @pl.when(wid == _NTILES - 1)
    def _():
        rem = -(-(_NCHUNK - (_NTILES - 1) * _GPT) // 8) * 8
        pltpu.sync_copy(src2d_hbm.at[pl.ds(r0, rem)],
                        idx_s.at[pl.ds(0, rem)])
        pltpu.sync_copy(dst2d_hbm.at[pl.ds(r0, rem)],
                        idx_d.at[pl.ds(0, rem)])

    def issue_g(i, bs, bd, sg_s, sg_d):
        pltpu.async_copy(tsrc_hbm.at[idx_s.at[i]], bs, sg_s)
        pltpu.async_copy(tdst_hbm.at[idx_d.at[i]], bd, sg_d)

    def wait_g(bs, bd, sg_s, sg_d):
        pltpu.make_async_copy(tsrc_hbm.at[idx_s.at[0]], bs, sg_s).wait()
        pltpu.make_async_copy(tdst_hbm.at[idx_d.at[0]], bd, sg_d).wait()

    def add_epre(bs, bd):
        @pl.loop(0, _CH)
        def _(r):
            @pl.loop(0, _H, step=16)
            def _(c):
                bs[r, pl.ds(c, 16)] = bs[r, pl.ds(c, 16)] + bd[r, pl.ds(c, 16)]

    def issue_w(i, bs, sw):
        base = (r0 + i) * _CH
        pltpu.async_copy(bs, g_hbm.at[pl.ds(base, _CH)], sw)

    def wait_w(bs, sw):
        pltpu.make_async_copy(bs, g_hbm.at[pl.ds(0, _CH)], sw).wait()

    issue_g(0, bs0, bd0, sgs0, sgd0)

    def step(j, carry):
        a = 2 * j

        wait_g(bs0, bd0, sgs0, sgd0)

        @pl.when(a + 1 < count)
        def _():
            @pl.when(j > 0)
            def _():
                wait_w(bs1, sw1)
            issue_g(a + 1, bs1, bd1, sgs1, sgd1)

        add_epre(bs0, bd0)
        issue_w(a, bs0, sw0)

        @pl.when(a + 1 < count)
        def _():
            wait_g(bs1, bd1, sgs1, sgd1)

            @pl.when(a + 2 < count)
            def _():
                wait_w(bs0, sw0)
                issue_g(a + 2, bs0, bd0, sgs0, sgd0)

            add_epre(bs1, bd1)
            issue_w(a + 1, bs1, sw1)

        return carry

    lax.fori_loop(0, (count + 1) // 2, step, 0)
    wait_w(bs0, sw0)

    @pl.when(count > 1)
    def _():
        wait_w(bs1, sw1)


@jax.jit
def _sc_gather(table_src, table_dst, src2d, dst2d):
    return pl.kernel(
        _gather_body,
        out_type=jax.ShapeDtypeStruct((_E, 256), jnp.float32),
        mesh=_mesh,
        scratch_types=[
            pltpu.VMEM((_GPT, _CH), jnp.int32),
            pltpu.VMEM((_GPT, _CH), jnp.int32),
            pltpu.VMEM((_CH, 256), jnp.float32),
            pltpu.VMEM((_CH, 256), jnp.float32),
            pltpu.VMEM((_CH, _H), jnp.float32),
            pltpu.VMEM((_CH, _H), jnp.float32),
            pltpu.SemaphoreType.DMA,
            pltpu.SemaphoreType.DMA,
            pltpu.SemaphoreType.DMA,
            pltpu.SemaphoreType.DMA,
            pltpu.SemaphoreType.DMA,
            pltpu.SemaphoreType.DMA,
        ],
    )(table_src, table_dst, src2d, dst2d)


_HSPT = 80


def _scatter_phase(d_hbm, dst2d_hbm, acc, idx, c0, c1, sl0, sl1, r0, count):
    @pl.when(count >= _HSPT)
    def _():
        pltpu.sync_copy(dst2d_hbm.at[pl.ds(r0, _HSPT)], idx)

    @pl.when(jnp.logical_and(count > 0, count < _HSPT))
    def _():
        rem = -(-(_NCHUNK - 15 * _SPT - _HSPT) // 8) * 8
        pltpu.sync_copy(dst2d_hbm.at[pl.ds(r0, rem)], idx.at[pl.ds(0, rem)])

    def issue_l(i, cb, sl):
        base = (r0 + i) * _CH
        pltpu.async_copy(d_hbm.at[pl.ds(base, _CH)], cb, sl)

    def wait_l(cb, sl):
        pltpu.make_async_copy(d_hbm.at[pl.ds(0, _CH)], cb, sl).wait()

    @pl.when(count > 0)
    def _():
        issue_l(0, c0, sl0)

    def step(j, carry):
        a = 2 * j

        wait_l(c0, sl0)

        @pl.when(a + 1 < count)
        def _():
            issue_l(a + 1, c1, sl1)

        pltpu.sync_copy(c0, acc.at[idx.at[a]], add=True)

        @pl.when(a + 1 < count)
        def _():
            wait_l(c1, sl1)

            @pl.when(a + 2 < count)
            def _():
                issue_l(a + 2, c0, sl0)

            pltpu.sync_copy(c1, acc.at[idx.at[a + 1]], add=True)

        return carry

    lax.fori_loop(0, (count + 1) // 2, step, 0)


def _scatter_loop(d_hbm, dst2d_hbm, acc, idx, c0, c1, sl0, sl1, sid):
    r0 = sid * _SPT
    n = jnp.maximum(0, jnp.minimum(_SPT, _NCHUNK - r0))
    c1st = jnp.minimum(n, _HSPT)
    _scatter_phase(d_hbm, dst2d_hbm, acc, idx, c0, c1, sl0, sl1, r0, c1st)
    _scatter_phase(d_hbm, dst2d_hbm, acc, idx, c0, c1, sl0, sl1,
                   r0 + _HSPT, n - c1st)


def _scatter_body(msg_hbm, sig_hbm, dst2d_hbm, num_hbm, den_hbm,
                  acc, idx, c0, c1, sl0, sl1):
    cid = lax.axis_index("c")
    sid = lax.axis_index("s")

    _zero_vmem(c0)
    row0 = sid * (_NPAD // 16)

    @pl.loop(0, _NPAD // 16, step=_ZR)
    def _(k):
        pltpu.sync_copy(c0, acc.at[pl.ds(row0 + k, _ZR)])

    plsc.subcore_barrier()

    @pl.when(cid == 0)
    def _():
        _scatter_loop(msg_hbm, dst2d_hbm, acc, idx, c0, c1, sl0, sl1, sid)

    @pl.when(cid == 1)
    def _():
        _scatter_loop(sig_hbm, dst2d_hbm, acc, idx, c0, c1, sl0, sl1, sid)

    plsc.subcore_barrier()

    @pl.when(cid == 0)
    def _():
        @pl.loop(0, _NPAD // 16, step=_ZR)
        def _(k):
            pltpu.sync_copy(acc.at[pl.ds(row0 + k, _ZR)],
                            num_hbm.at[pl.ds(row0 + k, _ZR)])

    @pl.when(cid == 1)
    def _():
        @pl.loop(0, _NPAD // 16, step=_ZR)
        def _(k):
            pltpu.sync_copy(acc.at[pl.ds(row0 + k, _ZR)],
                            den_hbm.at[pl.ds(row0 + k, _ZR)])


@jax.jit
def _sc_scatter(msg, sig, dst2d):
    return pl.kernel(
        _scatter_body,
        out_type=[jax.ShapeDtypeStruct((_NPAD, _H), jnp.float32),
                  jax.ShapeDtypeStruct((_NPAD, _H), jnp.float32)],
        mesh=_mesh,
        scratch_types=[
            pltpu.VMEM_SHARED((_NPAD, _H), jnp.float32),
            pltpu.VMEM((_HSPT, _CH), jnp.int32),
            pltpu.VMEM((_CH, _H), jnp.float32),
            pltpu.VMEM((_CH, _H), jnp.float32),
            pltpu.SemaphoreType.DMA,
            pltpu.SemaphoreType.DMA,
        ],
    )(msg, sig, dst2d)


def _edgewise_body(g_ref, c_ref, ein_ref, sn_ref,
                   enew_ref, sig_ref, msg_ref):
    epre = g_ref[:, :_H]
    bh_src = g_ref[:, _H:]
    ce = jnp.dot(ein_ref[...], c_ref[...],
                 preferred_element_type=jnp.float32)
    e_hat = epre + ce
    sig = jax.nn.sigmoid(e_hat)
    sig_ref[...] = sig
    msg_ref[...] = sig * bh_src
    enew_ref[...] = ein_ref[...] + jax.nn.relu(e_hat * sn_ref[...])


def _edgewise(g, C, e_in, snorm_e):
    n_blk = _E // _EBLK
    spec = pl.BlockSpec((_EBLK, _H), lambda i: (i, 0))
    spec2 = pl.BlockSpec((_EBLK, 256), lambda i: (i, 0))
    spec1 = pl.BlockSpec((_EBLK, 1), lambda i: (i, 0))
    specw = pl.BlockSpec((_H, _H), lambda i: (0, 0))
    out_shapes = [jax.ShapeDtypeStruct((_E, _H), jnp.float32)] * 3
    return pl.pallas_call(
        _edgewise_body,
        grid=(n_blk,),
        in_specs=[spec2, specw, spec, spec1],
        out_specs=[spec, spec, spec],
        out_shape=out_shapes,
    )(g, C, e_in, snorm_e)


def kernel(h, e, snorm_n, snorm_e, W_emb_h, W_emb_e, W_layers, W_ro, W_pred,
           b_pred, edge_index):
    pad = 2504 * _CH - _E
    src2d = jnp.pad(edge_index[0], (0, pad)).reshape(2504, _CH)
    dst2d = jnp.pad(edge_index[1], (0, pad)).reshape(2504, _CH)
    h = h @ W_emb_h
    e = e @ W_emb_e
    for l in range(4):
        h_in = h
        e_in = e
        A, B, C, Dw, Ew = (W_layers[l, i] for i in range(5))
        Ah = h @ A
        Bh = h @ B
        Dh = h @ Dw
        Eh = h @ Ew
        table_src = jnp.concatenate([Eh, Bh], axis=1)
        g = _sc_gather(table_src, Dh, src2d, dst2d)
        e_new, sig, msg = _edgewise(g, C, e_in, snorm_e)
        num, den = _sc_scatter(msg, sig, dst2d)
        num = num[:_N]
        den = den[:_N]
        h_new = jax.nn.relu((Ah + num / (den + 1e-6)) * snorm_n)
        e = e_new
        h = h_in + h_new
    hro = h @ W_ro
    hg = jnp.sum(hro, axis=0, keepdims=True)
    return hg @ W_pred + b_pred
